# baseline (speedup 1.0000x reference)
import numpy as np
import jax
import jax.numpy as jnp
from jax.experimental import pallas as pl
from jax.experimental.pallas import tpu as pltpu

H = 224
W = 224
T = 16
F = 8
C = 64
K = T * F
HALO = 4
HP = H + 2 * HALO
CPAD = 8
WP = W + CPAD
R = 56
NB = H // R
NR = R + 2 * HALO
H2 = H // 2
NR2 = NR // 2
P2 = NR2 * WP
HP2 = HP // 2


def _dinv_const():
    vp = np.zeros(HP, np.float64)
    r = np.arange(HP) - HALO
    inside = (r >= 0) & (r < H)
    vp[inside] = 1 + (r[inside] > 0) + (r[inside] < H - 1)
    hp = np.zeros(WP, np.float64)
    c = np.arange(WP) - CPAD
    insc = (c >= 0) & (c < W)
    hp[insc] = 1 + (c[insc] > 0) + (c[insc] < W - 1)
    deg = vp[:, None] * hp[None, :] + 1.0
    dinv = np.where(inside[:, None] & insc[None, :], 1.0 / np.sqrt(deg), 0.0)
    return dinv.astype(np.float32)


_DINV2D = _dinv_const()
_DINVP = np.concatenate(
    [
        np.broadcast_to(_DINV2D[0::2].reshape(HP2 * WP, 1), (HP2 * WP, C)),
        np.broadcast_to(_DINV2D[1::2].reshape(HP2 * WP, 1), (HP2 * WP, C)),
    ],
    axis=1,
).astype(np.float32)


def _shift(a, k):
    return jnp.concatenate([a[-k:], a[:-k]], axis=0)


def _body(x_hbm, dv_hbm, dv2d, w0, b0, wr0, br0, wr1, br1, w4, b4, out_ref,
          xblk, dvblk, sem_x, sem_d):
    i = pl.program_id(0)
    slot = jax.lax.rem(i, 2)

    def xcopy(blk, s, start):
        def mk(src_q0, dst_q0, m):
            return pltpu.make_async_copy(
                x_hbm.at[pl.ds(src_q0, m), :, :],
                xblk.at[s, pl.ds(dst_q0, m), pl.ds(CPAD, W), :],
                sem_x.at[s])

        q0 = jnp.maximum(blk * (R // 2) - HALO // 2, 0)

        @pl.when(blk == 0)
        def _():
            cp = mk(0, HALO // 2, NR2 - HALO // 2)
            cp.start() if start else cp.wait()

        @pl.when(jnp.logical_and(blk > 0, blk < NB - 1))
        def _():
            cp = mk(q0, 0, NR2)
            cp.start() if start else cp.wait()

        @pl.when(blk == NB - 1)
        def _():
            cp = mk(q0, 0, NR2 - HALO // 2)
            cp.start() if start else cp.wait()

    def dcopy(blk, s, start):
        cp = pltpu.make_async_copy(
            dv_hbm.at[pl.ds(blk * (R // 2) * WP, P2), :], dvblk.at[s],
            sem_d.at[s])
        cp.start() if start else cp.wait()

    @pl.when(i == 0)
    def _():
        zc = jnp.zeros((NR2, CPAD, 2 * K), jnp.float32)
        xblk[0, :, 0:CPAD, :] = zc
        xblk[1, :, 0:CPAD, :] = zc
        xblk[0, 0:HALO // 2, pl.ds(CPAD, W), :] = jnp.zeros(
            (HALO // 2, W, 2 * K), jnp.float32)
        xcopy(0, 0, True)
        dcopy(0, 0, True)

    @pl.when(i + 1 < NB)
    def _():
        xcopy(i + 1, 1 - slot, True)
        dcopy(i + 1, 1 - slot, True)

    xcopy(i, slot, False)
    dcopy(i, slot, False)

    dinv = dvblk[slot]
    lmask = jax.lax.broadcasted_iota(jnp.int32, (P2, 2 * C), 1) < C

    def agg(xw, brow):
        s = dinv * xw
        csum = s + _shift(s, 1) + _shift(s, -1)
        t = jnp.concatenate([csum[:, C:], csum[:, :C]], axis=1)
        up = jnp.where(lmask, _shift(t, WP), t)
        dn = jnp.where(lmask, t, _shift(t, -WP))
        box = csum + up + dn
        return dinv * (box + s) + brow

    def gcn(inp, w, brow):
        return agg(jnp.dot(inp, w, preferred_element_type=jnp.float32), brow)

    x = jnp.reshape(xblk[slot], (P2, 2 * K))
    h0 = gcn(x, w0[...], b0[...])
    h1 = gcn(jax.nn.relu(h0), wr0[...], br0[...])
    h2 = gcn(jax.nn.relu(h1), wr1[...], br1[...])
    ls = jax.nn.relu(h0 + h1 + h2)

    f4 = jnp.dot(ls, w4[...], preferred_element_type=jnp.float32)
    e2 = f4[:, 0:1].reshape(NR2, WP)
    o2 = f4[:, 1:2].reshape(NR2, WP)
    f2 = jnp.stack([e2, o2], axis=1).reshape(NR, WP)
    dv2 = dv2d[pl.ds(i * R, NR), :]
    s3 = dv2 * f2
    c3 = s3 \
        + jnp.concatenate([s3[:, -1:], s3[:, :-1]], axis=1) \
        + jnp.concatenate([s3[:, 1:], s3[:, :1]], axis=1)
    b3 = c3 \
        + jnp.concatenate([c3[-1:], c3[:-1]], axis=0) \
        + jnp.concatenate([c3[1:], c3[:1]], axis=0)
    h4 = dv2 * (b3 + s3) + b4[...]
    out_ref[...] = h4[HALO:HALO + R, :]


def _bd(w):
    a, b = w.shape
    z = jnp.zeros((a, b), w.dtype)
    return jnp.concatenate(
        [jnp.concatenate([w, z], axis=1), jnp.concatenate([z, w], axis=1)],
        axis=0)


def kernel(l_input, y, W0, b0, Wr0, br0, Wr1, br1, W4, b4):
    x4 = jnp.transpose(
        l_input.reshape(1, T, H2, 2, W, F), (0, 2, 4, 3, 1, 5)
    ).reshape(H2, W, 2 * K)
    W0p = jnp.transpose(W0.reshape(F, T, C), (1, 0, 2)).reshape(K, C)
    dinv = jnp.asarray(_DINVP)
    dinv2d = jnp.asarray(_DINV2D)

    out = pl.pallas_call(
        _body,
        grid=(NB,),
        in_specs=[
            pl.BlockSpec(memory_space=pltpu.MemorySpace.HBM),
            pl.BlockSpec(memory_space=pltpu.MemorySpace.HBM),
            pl.BlockSpec((HP, WP), lambda i: (0, 0)),
            pl.BlockSpec((2 * K, 2 * C), lambda i: (0, 0)),
            pl.BlockSpec((1, 2 * C), lambda i: (0, 0)),
            pl.BlockSpec((2 * C, 2 * C), lambda i: (0, 0)),
            pl.BlockSpec((1, 2 * C), lambda i: (0, 0)),
            pl.BlockSpec((2 * C, 2 * C), lambda i: (0, 0)),
            pl.BlockSpec((1, 2 * C), lambda i: (0, 0)),
            pl.BlockSpec((2 * C, 2), lambda i: (0, 0)),
            pl.BlockSpec((1, 1), lambda i: (0, 0)),
        ],
        out_specs=pl.BlockSpec((R, WP), lambda i: (i, 0)),
        out_shape=jax.ShapeDtypeStruct((H, WP), jnp.float32),
        scratch_shapes=[
            pltpu.VMEM((2, NR2, WP, 2 * K), jnp.float32),
            pltpu.VMEM((2, P2, 2 * C), jnp.float32),
            pltpu.SemaphoreType.DMA((2,)),
            pltpu.SemaphoreType.DMA((2,)),
        ],
    )(x4, dinv, dinv2d, _bd(W0p), jnp.tile(b0.reshape(1, C), (1, 2)),
      _bd(Wr0), jnp.tile(br0.reshape(1, C), (1, 2)),
      _bd(Wr1), jnp.tile(br1.reshape(1, C), (1, 2)),
      _bd(W4), b4.reshape(1, 1))

    out = out[:, CPAD:CPAD + W]
    return out.reshape(1, 1, H, W, 1)


# speedup vs baseline: 1.0478x; 1.0478x over previous
import numpy as np
import jax
import jax.numpy as jnp
from jax.experimental import pallas as pl
from jax.experimental.pallas import tpu as pltpu

H = 224
W = 224
T = 16
F = 8
C = 64
K = T * F
HALO = 4
HP = H + 2 * HALO
CPAD = 8
WP = W + CPAD
R = 56
NB = H // R
NR = R + 2 * HALO
H2 = H // 2
NR2 = NR // 2
P2 = NR2 * WP
HP2 = HP // 2


def _dinv_const():
    vp = np.zeros(HP, np.float64)
    r = np.arange(HP) - HALO
    inside = (r >= 0) & (r < H)
    vp[inside] = 1 + (r[inside] > 0) + (r[inside] < H - 1)
    hp = np.zeros(WP, np.float64)
    c = np.arange(WP) - CPAD
    insc = (c >= 0) & (c < W)
    hp[insc] = 1 + (c[insc] > 0) + (c[insc] < W - 1)
    deg = vp[:, None] * hp[None, :] + 1.0
    dinv = np.where(inside[:, None] & insc[None, :], 1.0 / np.sqrt(deg), 0.0)
    return dinv.astype(np.float32)


_DINV2D = _dinv_const()
_DINVP = np.concatenate(
    [
        np.broadcast_to(_DINV2D[0::2].reshape(HP2 * WP, 1), (HP2 * WP, C)),
        np.broadcast_to(_DINV2D[1::2].reshape(HP2 * WP, 1), (HP2 * WP, C)),
    ],
    axis=1,
).astype(np.float32)


def _shift(a, k):
    return jnp.concatenate([a[-k:], a[:-k]], axis=0)


def _body(x_hbm, dv_hbm, dv2d, w0, b0, wr0, br0, wr1, br1, w4, b4, out_ref,
          xblk, dvblk, sem_x, sem_d):
    i = pl.program_id(0)
    slot = jax.lax.rem(i, 2)

    def xcopy(blk, s, start):
        def mk(src_q0, dst_q0, m):
            return pltpu.make_async_copy(
                x_hbm.at[pl.ds(src_q0, m), :, :],
                xblk.at[s, pl.ds(dst_q0, m), pl.ds(CPAD, W), :],
                sem_x.at[s])

        q0 = jnp.maximum(blk * (R // 2) - HALO // 2, 0)

        @pl.when(blk == 0)
        def _():
            cp = mk(0, HALO // 2, NR2 - HALO // 2)
            cp.start() if start else cp.wait()

        @pl.when(jnp.logical_and(blk > 0, blk < NB - 1))
        def _():
            cp = mk(q0, 0, NR2)
            cp.start() if start else cp.wait()

        @pl.when(blk == NB - 1)
        def _():
            cp = mk(q0, 0, NR2 - HALO // 2)
            cp.start() if start else cp.wait()

    def dcopy(blk, s, start):
        cp = pltpu.make_async_copy(
            dv_hbm.at[pl.ds(blk * (R // 2) * WP, P2), :], dvblk.at[s],
            sem_d.at[s])
        cp.start() if start else cp.wait()

    @pl.when(i == 0)
    def _():
        zc = jnp.zeros((NR2, CPAD, 2 * K), jnp.bfloat16)
        xblk[0, :, 0:CPAD, :] = zc
        xblk[1, :, 0:CPAD, :] = zc
        xblk[0, 0:HALO // 2, pl.ds(CPAD, W), :] = jnp.zeros(
            (HALO // 2, W, 2 * K), jnp.bfloat16)
        xcopy(0, 0, True)
        dcopy(0, 0, True)

    @pl.when(i + 1 < NB)
    def _():
        xcopy(i + 1, 1 - slot, True)
        dcopy(i + 1, 1 - slot, True)

    xcopy(i, slot, False)
    dcopy(i, slot, False)

    dinv = dvblk[slot]
    lmask = jax.lax.broadcasted_iota(jnp.int32, (P2, 2 * C), 1) < C

    def agg(xw, brow):
        s = dinv * xw
        csum = s + _shift(s, 1) + _shift(s, -1)
        t = jnp.concatenate([csum[:, C:], csum[:, :C]], axis=1)
        up = jnp.where(lmask, _shift(t, WP), t)
        dn = jnp.where(lmask, t, _shift(t, -WP))
        box = csum + up + dn
        return dinv * (box + s) + brow

    def gcn(inp, w, brow):
        return agg(jnp.dot(inp, w, preferred_element_type=jnp.float32), brow)

    x = jnp.reshape(xblk[slot], (P2, 2 * K))
    h0 = gcn(x, w0[...], b0[...])
    h1 = gcn(jax.nn.relu(h0), wr0[...], br0[...])
    h2 = gcn(jax.nn.relu(h1), wr1[...], br1[...])
    ls = jax.nn.relu(h0 + h1 + h2)

    f4 = jnp.dot(ls, w4[...], preferred_element_type=jnp.float32)
    e2 = f4[:, 0:1].reshape(NR2, WP)
    o2 = f4[:, 1:2].reshape(NR2, WP)
    f2 = jnp.stack([e2, o2], axis=1).reshape(NR, WP)
    dv2 = dv2d[pl.ds(i * R, NR), :]
    s3 = dv2 * f2
    c3 = s3 \
        + jnp.concatenate([s3[:, -1:], s3[:, :-1]], axis=1) \
        + jnp.concatenate([s3[:, 1:], s3[:, :1]], axis=1)
    b3 = c3 \
        + jnp.concatenate([c3[-1:], c3[:-1]], axis=0) \
        + jnp.concatenate([c3[1:], c3[:1]], axis=0)
    h4 = dv2 * (b3 + s3) + b4[...]
    out_ref[...] = h4[HALO:HALO + R, :]


def _bd(w):
    a, b = w.shape
    z = jnp.zeros((a, b), w.dtype)
    return jnp.concatenate(
        [jnp.concatenate([w, z], axis=1), jnp.concatenate([z, w], axis=1)],
        axis=0)


def kernel(l_input, y, W0, b0, Wr0, br0, Wr1, br1, W4, b4):
    x4 = jnp.transpose(
        l_input.astype(jnp.bfloat16).reshape(1, T, H2, 2, W, F),
        (0, 2, 4, 3, 1, 5)
    ).reshape(H2, W, 2 * K)
    W0p = jnp.transpose(W0.reshape(F, T, C), (1, 0, 2)).reshape(K, C)
    dinv = jnp.asarray(_DINVP)
    dinv2d = jnp.asarray(_DINV2D)

    out = pl.pallas_call(
        _body,
        grid=(NB,),
        in_specs=[
            pl.BlockSpec(memory_space=pltpu.MemorySpace.HBM),
            pl.BlockSpec(memory_space=pltpu.MemorySpace.HBM),
            pl.BlockSpec((HP, WP), lambda i: (0, 0)),
            pl.BlockSpec((2 * K, 2 * C), lambda i: (0, 0)),
            pl.BlockSpec((1, 2 * C), lambda i: (0, 0)),
            pl.BlockSpec((2 * C, 2 * C), lambda i: (0, 0)),
            pl.BlockSpec((1, 2 * C), lambda i: (0, 0)),
            pl.BlockSpec((2 * C, 2 * C), lambda i: (0, 0)),
            pl.BlockSpec((1, 2 * C), lambda i: (0, 0)),
            pl.BlockSpec((2 * C, 2), lambda i: (0, 0)),
            pl.BlockSpec((1, 1), lambda i: (0, 0)),
        ],
        out_specs=pl.BlockSpec((R, WP), lambda i: (i, 0)),
        out_shape=jax.ShapeDtypeStruct((H, WP), jnp.float32),
        scratch_shapes=[
            pltpu.VMEM((2, NR2, WP, 2 * K), jnp.bfloat16),
            pltpu.VMEM((2, P2, 2 * C), jnp.float32),
            pltpu.SemaphoreType.DMA((2,)),
            pltpu.SemaphoreType.DMA((2,)),
        ],
    )(x4, dinv, dinv2d, _bd(W0p).astype(jnp.bfloat16),
      jnp.tile(b0.reshape(1, C), (1, 2)),
      _bd(Wr0), jnp.tile(br0.reshape(1, C), (1, 2)),
      _bd(Wr1), jnp.tile(br1.reshape(1, C), (1, 2)),
      _bd(W4), b4.reshape(1, 1))

    out = out[:, CPAD:CPAD + W]
    return out.reshape(1, 1, H, W, 1)


# speedup vs baseline: 1.0572x; 1.0090x over previous
import numpy as np
import jax
import jax.numpy as jnp
from jax.experimental import pallas as pl
from jax.experimental.pallas import tpu as pltpu

H = 224
W = 224
T = 16
F = 8
C = 64
K = T * F
HALO = 4
HP = H + 2 * HALO
WP = W + 2 * HALO
R = 56
NB = H // R
NR = R + 2 * HALO
P = NR * WP


def _dinv_const():
    vp = np.zeros(HP, np.float64)
    r = np.arange(HP) - HALO
    inside = (r >= 0) & (r < H)
    vp[inside] = 1 + (r[inside] > 0) + (r[inside] < H - 1)
    hp = np.zeros(WP, np.float64)
    c = np.arange(WP) - HALO
    insc = (c >= 0) & (c < W)
    hp[insc] = 1 + (c[insc] > 0) + (c[insc] < W - 1)
    deg = vp[:, None] * hp[None, :] + 1.0
    dinv = np.where(inside[:, None] & insc[None, :], 1.0 / np.sqrt(deg), 0.0)
    return dinv.astype(np.float32)


_DINV2D = _dinv_const()
_DINV = np.ascontiguousarray(
    np.broadcast_to(_DINV2D.reshape(HP * WP, 1), (HP * WP, C))
).astype(np.float32)


def _shift(a, k):
    return jnp.concatenate([a[-k:], a[:-k]], axis=0)


def _body(x_hbm, dv_hbm, dv2d, w0, b0, wr0, br0, wr1, br1, w4, b4, out_ref,
          xblk, dvblk, sem_x, sem_d):
    i = pl.program_id(0)
    slot = jax.lax.rem(i, 2)

    def xcopy(blk, s, start):
        def mk(src_r0, dst_r0, nrows):
            return pltpu.make_async_copy(
                x_hbm.at[pl.ds(src_r0, nrows), :, :],
                xblk.at[s, pl.ds(dst_r0, nrows), pl.ds(HALO, W), :],
                sem_x.at[s])

        @pl.when(blk == 0)
        def _():
            cp = mk(0, HALO, NR - HALO)
            cp.start() if start else cp.wait()

        r0 = jnp.maximum(blk * R - HALO, 0)

        @pl.when(jnp.logical_and(blk > 0, blk < NB - 1))
        def _():
            cp = mk(r0, 0, NR)
            cp.start() if start else cp.wait()

        @pl.when(blk == NB - 1)
        def _():
            cp = mk(r0, 0, NR - HALO)
            cp.start() if start else cp.wait()

    def dcopy(blk, s, start):
        cp = pltpu.make_async_copy(
            dv_hbm.at[pl.ds(blk * R * WP, P), :], dvblk.at[s], sem_d.at[s])
        cp.start() if start else cp.wait()

    @pl.when(i == 0)
    def _():
        xblk[0, :, 0:HALO, :] = jnp.zeros((NR, HALO, K), jnp.float32)
        xblk[1, :, 0:HALO, :] = jnp.zeros((NR, HALO, K), jnp.float32)
        xblk[0, :, HALO + W:WP, :] = jnp.zeros((NR, HALO, K), jnp.float32)
        xblk[1, :, HALO + W:WP, :] = jnp.zeros((NR, HALO, K), jnp.float32)
        xblk[0, 0:HALO, pl.ds(HALO, W), :] = jnp.zeros((HALO, W, K), jnp.float32)
        xcopy(0, 0, True)
        dcopy(0, 0, True)

    @pl.when(i + 1 < NB)
    def _():
        xcopy(i + 1, 1 - slot, True)
        dcopy(i + 1, 1 - slot, True)

    xcopy(i, slot, False)
    dcopy(i, slot, False)

    dinv = dvblk[slot]

    def agg(xw, brow):
        s = dinv * xw
        csum = s + _shift(s, 1) + _shift(s, -1)
        box = csum + _shift(csum, WP) + _shift(csum, -WP)
        return dinv * (box + s) + brow

    def gcn(inp, w, brow):
        return agg(jnp.dot(inp, w, preferred_element_type=jnp.float32), brow)

    x = jnp.reshape(xblk[slot], (P, K))
    h0 = gcn(x, w0[...], b0[...])
    h1 = gcn(jax.nn.relu(h0), wr0[...], br0[...])
    h2 = gcn(jax.nn.relu(h1), wr1[...], br1[...])
    ls = jax.nn.relu(h0 + h1 + h2)

    xw4 = jnp.dot(ls, w4[...], preferred_element_type=jnp.float32)
    f2 = xw4.reshape(NR, WP)
    dv2 = dv2d[pl.ds(i * R, NR), :]
    s3 = dv2 * f2
    c3 = s3 \
        + jnp.concatenate([s3[:, -1:], s3[:, :-1]], axis=1) \
        + jnp.concatenate([s3[:, 1:], s3[:, :1]], axis=1)
    b3 = c3 \
        + jnp.concatenate([c3[-1:], c3[:-1]], axis=0) \
        + jnp.concatenate([c3[1:], c3[:1]], axis=0)
    h4 = dv2 * (b3 + s3) + b4[...]
    out_ref[...] = h4[HALO:HALO + R, :]


def kernel(l_input, y, W0, b0, Wr0, br0, Wr1, br1, W4, b4):
    x3 = jnp.transpose(l_input, (0, 2, 3, 1, 4)).reshape(H, W, K)
    W0p = jnp.transpose(W0.reshape(F, T, C), (1, 0, 2)).reshape(K, C)
    dinv = jnp.asarray(_DINV)
    dinv2d = jnp.asarray(_DINV2D)

    out = pl.pallas_call(
        _body,
        grid=(NB,),
        in_specs=[
            pl.BlockSpec(memory_space=pltpu.MemorySpace.HBM),
            pl.BlockSpec(memory_space=pltpu.MemorySpace.HBM),
            pl.BlockSpec((HP, WP), lambda i: (0, 0)),
            pl.BlockSpec((K, C), lambda i: (0, 0)),
            pl.BlockSpec((1, C), lambda i: (0, 0)),
            pl.BlockSpec((C, C), lambda i: (0, 0)),
            pl.BlockSpec((1, C), lambda i: (0, 0)),
            pl.BlockSpec((C, C), lambda i: (0, 0)),
            pl.BlockSpec((1, C), lambda i: (0, 0)),
            pl.BlockSpec((C, 1), lambda i: (0, 0)),
            pl.BlockSpec((1, 1), lambda i: (0, 0)),
        ],
        out_specs=pl.BlockSpec((R, WP), lambda i: (i, 0)),
        out_shape=jax.ShapeDtypeStruct((H, WP), jnp.float32),
        scratch_shapes=[
            pltpu.VMEM((2, NR, WP, K), jnp.float32),
            pltpu.VMEM((2, P, C), jnp.float32),
            pltpu.SemaphoreType.DMA((2,)),
            pltpu.SemaphoreType.DMA((2,)),
        ],
    )(x3, dinv, dinv2d, W0p, b0.reshape(1, C), Wr0, br0.reshape(1, C),
      Wr1, br1.reshape(1, C), W4, b4.reshape(1, 1))

    out = out[:, HALO:HALO + W]
    return out.reshape(1, 1, H, W, 1)


# speedup vs baseline: 1.1880x; 1.1237x over previous
import numpy as np
import jax
import jax.numpy as jnp
from jax.experimental import pallas as pl
from jax.experimental.pallas import tpu as pltpu

H = 224
W = 224
T = 16
F = 8
C = 64
K = T * F
HALO = 4
HP = H + 2 * HALO
CPAD = 8
WP = W + CPAD
R = 56
NB = H // R
NR = R + 2 * HALO
NR2 = NR // 2
P2 = NR2 * WP


def _dinv_const():
    vp = np.zeros(HP, np.float64)
    r = np.arange(HP) - HALO
    inside = (r >= 0) & (r < H)
    vp[inside] = 1 + (r[inside] > 0) + (r[inside] < H - 1)
    hp = np.zeros(WP, np.float64)
    c = np.arange(WP) - CPAD
    insc = (c >= 0) & (c < W)
    hp[insc] = 1 + (c[insc] > 0) + (c[insc] < W - 1)
    deg = vp[:, None] * hp[None, :] + 1.0
    dinv = np.where(inside[:, None] & insc[None, :], 1.0 / np.sqrt(deg), 0.0)
    return dinv.astype(np.float32)


_DINV2D = _dinv_const()


def _dinv_blocked():
    blocks = []
    for i in range(NB):
        rows = _DINV2D[i * R:i * R + NR]
        a = np.broadcast_to(rows[:NR2].reshape(P2, 1), (P2, C))
        b = np.broadcast_to(rows[NR2:].reshape(P2, 1), (P2, C))
        blocks.append(np.concatenate([a, b], axis=1))
    return np.ascontiguousarray(np.stack(blocks)).astype(np.float32)


_DINVP = _dinv_blocked()


def _shift(a, k):
    return jnp.concatenate([a[-k:], a[:-k]], axis=0)


def _body(x_hbm, dv_hbm, dv2d, w0, b0, wr0, br0, wr1, br1, w4, b4, out_ref,
          xblk, dvblk, sem_x, sem_d):
    i = pl.program_id(0)
    slot = jax.lax.rem(i, 2)

    def xcopy(blk, s, start):
        def mk(src_r0, dst_q0, m, half):
            return pltpu.make_async_copy(
                x_hbm.at[pl.ds(src_r0, m), :, :],
                xblk.at[s, pl.ds(dst_q0, m), pl.ds(CPAD, W),
                        pl.ds(half * K, K)],
                sem_x.at[s])

        r0 = jnp.maximum(blk * R - HALO, 0)

        @pl.when(blk == 0)
        def _():
            for cp in (mk(0, HALO, NR2 - HALO, 0),
                       mk(NR2 - HALO, 0, NR2, 1)):
                cp.start() if start else cp.wait()

        @pl.when(jnp.logical_and(blk > 0, blk < NB - 1))
        def _():
            for cp in (mk(r0, 0, NR2, 0),
                       mk(r0 + NR2, 0, NR2, 1)):
                cp.start() if start else cp.wait()

        @pl.when(blk == NB - 1)
        def _():
            for cp in (mk(r0, 0, NR2, 0),
                       mk(r0 + NR2, 0, NR2 - HALO, 1)):
                cp.start() if start else cp.wait()

    def dcopy(blk, s, start):
        cp = pltpu.make_async_copy(
            dv_hbm.at[pl.ds(blk * P2, P2), :], dvblk.at[s], sem_d.at[s])
        cp.start() if start else cp.wait()

    @pl.when(i == 0)
    def _():
        zc = jnp.zeros((NR2, CPAD, 2 * K), jnp.float32)
        xblk[0, :, 0:CPAD, :] = zc
        xblk[1, :, 0:CPAD, :] = zc
        xblk[0, 0:HALO, pl.ds(CPAD, W), 0:K] = jnp.zeros(
            (HALO, W, K), jnp.float32)
        xcopy(0, 0, True)
        dcopy(0, 0, True)

    @pl.when(i + 1 < NB)
    def _():
        xcopy(i + 1, 1 - slot, True)
        dcopy(i + 1, 1 - slot, True)

    xcopy(i, slot, False)
    dcopy(i, slot, False)

    dinv = dvblk[slot]
    qio = jax.lax.broadcasted_iota(jnp.int32, (P2, 2 * C), 0)
    lane = jax.lax.broadcasted_iota(jnp.int32, (P2, 2 * C), 1)
    lane_a = lane < C
    fix_up = jnp.logical_and(qio < WP, jnp.logical_not(lane_a))
    fix_dn = jnp.logical_and(qio >= P2 - WP, lane_a)

    def agg(xw, brow):
        s = dinv * xw
        csum = s + _shift(s, 1) + _shift(s, -1)
        t = jnp.concatenate([csum[:, C:], csum[:, :C]], axis=1)
        up = jnp.where(fix_up, _shift(t, WP), _shift(csum, WP))
        dn = jnp.where(fix_dn, _shift(t, -WP), _shift(csum, -WP))
        box = csum + up + dn
        return dinv * (box + s) + brow

    def gcn(inp, w, brow):
        return agg(jnp.dot(inp, w, preferred_element_type=jnp.float32), brow)

    x = jnp.reshape(xblk[slot], (P2, 2 * K))
    h0 = gcn(x, w0[...], b0[...])
    h1 = gcn(jax.nn.relu(h0), wr0[...], br0[...])
    h2 = gcn(jax.nn.relu(h1), wr1[...], br1[...])
    ls = jax.nn.relu(h0 + h1 + h2)

    f4 = jnp.dot(ls, w4[...], preferred_element_type=jnp.float32)
    f2 = jnp.concatenate(
        [f4[:, 0:1].reshape(NR2, WP), f4[:, 1:2].reshape(NR2, WP)], axis=0)
    dv2 = dv2d[pl.ds(i * R, NR), :]
    s3 = dv2 * f2
    c3 = s3 \
        + jnp.concatenate([s3[:, -1:], s3[:, :-1]], axis=1) \
        + jnp.concatenate([s3[:, 1:], s3[:, :1]], axis=1)
    b3 = c3 \
        + jnp.concatenate([c3[-1:], c3[:-1]], axis=0) \
        + jnp.concatenate([c3[1:], c3[:1]], axis=0)
    h4 = dv2 * (b3 + s3) + b4[...]
    out_ref[...] = h4[HALO:HALO + R, :]


def _bd(w):
    a, b = w.shape
    z = jnp.zeros((a, b), w.dtype)
    return jnp.concatenate(
        [jnp.concatenate([w, z], axis=1), jnp.concatenate([z, w], axis=1)],
        axis=0)


def kernel(l_input, y, W0, b0, Wr0, br0, Wr1, br1, W4, b4):
    x3 = jnp.transpose(l_input, (0, 2, 3, 1, 4)).reshape(H, W, K)
    W0p = jnp.transpose(W0.reshape(F, T, C), (1, 0, 2)).reshape(K, C)
    dinv = jnp.asarray(_DINVP).reshape(NB * P2, 2 * C)
    dinv2d = jnp.asarray(_DINV2D)

    out = pl.pallas_call(
        _body,
        grid=(NB,),
        in_specs=[
            pl.BlockSpec(memory_space=pltpu.MemorySpace.HBM),
            pl.BlockSpec(memory_space=pltpu.MemorySpace.HBM),
            pl.BlockSpec((HP, WP), lambda i: (0, 0)),
            pl.BlockSpec((2 * K, 2 * C), lambda i: (0, 0)),
            pl.BlockSpec((1, 2 * C), lambda i: (0, 0)),
            pl.BlockSpec((2 * C, 2 * C), lambda i: (0, 0)),
            pl.BlockSpec((1, 2 * C), lambda i: (0, 0)),
            pl.BlockSpec((2 * C, 2 * C), lambda i: (0, 0)),
            pl.BlockSpec((1, 2 * C), lambda i: (0, 0)),
            pl.BlockSpec((2 * C, 2), lambda i: (0, 0)),
            pl.BlockSpec((1, 1), lambda i: (0, 0)),
        ],
        out_specs=pl.BlockSpec((R, WP), lambda i: (i, 0)),
        out_shape=jax.ShapeDtypeStruct((H, WP), jnp.float32),
        scratch_shapes=[
            pltpu.VMEM((2, NR2, WP, 2 * K), jnp.float32),
            pltpu.VMEM((2, P2, 2 * C), jnp.float32),
            pltpu.SemaphoreType.DMA((2,)),
            pltpu.SemaphoreType.DMA((2,)),
        ],
    )(x3, dinv, dinv2d, _bd(W0p), jnp.tile(b0.reshape(1, C), (1, 2)),
      _bd(Wr0), jnp.tile(br0.reshape(1, C), (1, 2)),
      _bd(Wr1), jnp.tile(br1.reshape(1, C), (1, 2)),
      _bd(W4), b4.reshape(1, 1))

    out = out[:, CPAD:CPAD + W]
    return out.reshape(1, 1, H, W, 1)


# speedup vs baseline: 1.3897x; 1.1698x over previous
import numpy as np
import jax
import jax.numpy as jnp
from jax.experimental import pallas as pl
from jax.experimental.pallas import tpu as pltpu

H = 224
W = 224
T = 16
F = 8
C = 64
K = T * F
HALO = 4
HP = H + 2 * HALO
CPAD = 8
WP = W + CPAD
R = 56
NB = H // R
NR = R + 2 * HALO
NR2 = NR // 2
P2 = NR2 * WP


def _dinv_const():
    vp = np.zeros(HP, np.float64)
    r = np.arange(HP) - HALO
    inside = (r >= 0) & (r < H)
    vp[inside] = 1 + (r[inside] > 0) + (r[inside] < H - 1)
    hp = np.zeros(WP, np.float64)
    c = np.arange(WP) - CPAD
    insc = (c >= 0) & (c < W)
    hp[insc] = 1 + (c[insc] > 0) + (c[insc] < W - 1)
    deg = vp[:, None] * hp[None, :] + 1.0
    dinv = np.where(inside[:, None] & insc[None, :], 1.0 / np.sqrt(deg), 0.0)
    return dinv.astype(np.float32)


_DINV2D = _dinv_const()


def _dinv_blocked():
    blocks = []
    for i in range(NB):
        rows = _DINV2D[i * R:i * R + NR]
        a = np.broadcast_to(rows[:NR2].reshape(P2, 1), (P2, C))
        b = np.broadcast_to(rows[NR2:].reshape(P2, 1), (P2, C))
        blocks.append(np.concatenate([a, b], axis=1))
    return np.ascontiguousarray(np.stack(blocks)).astype(np.float32)


_DINVP = _dinv_blocked()


def _shift(a, k):
    return jnp.concatenate([a[-k:], a[:-k]], axis=0)


def _body(x_hbm, dv_hbm, dv2d, w0, b0, wr0, br0, wr1, br1, w4, b4, out_ref,
          xblk, dvblk, sem_x, sem_d):
    i = pl.program_id(0)
    slot = jax.lax.rem(i, 2)

    def xcopy(blk, s, start):
        def mk(src_r0, dst_q0, m, half):
            return pltpu.make_async_copy(
                x_hbm.at[pl.ds(src_r0, m), :, :],
                xblk.at[s, pl.ds(dst_q0, m), pl.ds(CPAD, W),
                        pl.ds(half * K, K)],
                sem_x.at[s])

        r0 = jnp.maximum(blk * R - HALO, 0)

        @pl.when(blk == 0)
        def _():
            for cp in (mk(0, HALO, NR2 - HALO, 0),
                       mk(NR2 - HALO, 0, NR2, 1)):
                cp.start() if start else cp.wait()

        @pl.when(jnp.logical_and(blk > 0, blk < NB - 1))
        def _():
            for cp in (mk(r0, 0, NR2, 0),
                       mk(r0 + NR2, 0, NR2, 1)):
                cp.start() if start else cp.wait()

        @pl.when(blk == NB - 1)
        def _():
            for cp in (mk(r0, 0, NR2, 0),
                       mk(r0 + NR2, 0, NR2 - HALO, 1)):
                cp.start() if start else cp.wait()

    def dcopy(blk, s, start):
        cp = pltpu.make_async_copy(
            dv_hbm.at[pl.ds(blk * P2, P2), :], dvblk.at[s], sem_d.at[s])
        cp.start() if start else cp.wait()

    @pl.when(i == 0)
    def _():
        zc = jnp.zeros((NR2, CPAD, 2 * K), jnp.bfloat16)
        xblk[0, :, 0:CPAD, :] = zc
        xblk[1, :, 0:CPAD, :] = zc
        xblk[0, 0:HALO, pl.ds(CPAD, W), 0:K] = jnp.zeros(
            (HALO, W, K), jnp.bfloat16)
        xcopy(0, 0, True)
        dcopy(0, 0, True)

    @pl.when(i + 1 < NB)
    def _():
        xcopy(i + 1, 1 - slot, True)
        dcopy(i + 1, 1 - slot, True)

    xcopy(i, slot, False)
    dcopy(i, slot, False)

    dinv = dvblk[slot]
    qio = jax.lax.broadcasted_iota(jnp.int32, (P2, 2 * C), 0)
    lane = jax.lax.broadcasted_iota(jnp.int32, (P2, 2 * C), 1)
    lane_a = lane < C
    fix_up = jnp.logical_and(qio < WP, jnp.logical_not(lane_a))
    fix_dn = jnp.logical_and(qio >= P2 - WP, lane_a)

    def agg(xw, brow):
        s = dinv * xw
        csum = s + _shift(s, 1) + _shift(s, -1)
        t = jnp.concatenate([csum[:, C:], csum[:, :C]], axis=1)
        up = jnp.where(fix_up, _shift(t, WP), _shift(csum, WP))
        dn = jnp.where(fix_dn, _shift(t, -WP), _shift(csum, -WP))
        box = csum + up + dn
        return dinv * (box + s) + brow

    def gcn(inp, w, brow):
        return agg(jnp.dot(inp, w, preferred_element_type=jnp.float32), brow)

    x = jnp.reshape(xblk[slot], (P2, 2 * K))
    h0 = gcn(x, w0[...], b0[...])
    h1 = gcn(jax.nn.relu(h0), wr0[...], br0[...])
    h2 = gcn(jax.nn.relu(h1), wr1[...], br1[...])
    ls = jax.nn.relu(h0 + h1 + h2)

    f4 = jnp.dot(ls, w4[...], preferred_element_type=jnp.float32)
    f2 = jnp.concatenate(
        [f4[:, 0:1].reshape(NR2, WP), f4[:, 1:2].reshape(NR2, WP)], axis=0)
    dv2 = dv2d[pl.ds(i * R, NR), :]
    s3 = dv2 * f2
    c3 = s3 \
        + jnp.concatenate([s3[:, -1:], s3[:, :-1]], axis=1) \
        + jnp.concatenate([s3[:, 1:], s3[:, :1]], axis=1)
    b3 = c3 \
        + jnp.concatenate([c3[-1:], c3[:-1]], axis=0) \
        + jnp.concatenate([c3[1:], c3[:1]], axis=0)
    h4 = dv2 * (b3 + s3) + b4[...]
    out_ref[...] = h4[HALO:HALO + R, :]


def _bd(w):
    a, b = w.shape
    z = jnp.zeros((a, b), w.dtype)
    return jnp.concatenate(
        [jnp.concatenate([w, z], axis=1), jnp.concatenate([z, w], axis=1)],
        axis=0)


def kernel(l_input, y, W0, b0, Wr0, br0, Wr1, br1, W4, b4):
    x3 = jnp.transpose(
        l_input.astype(jnp.bfloat16), (0, 2, 3, 1, 4)).reshape(H, W, K)
    W0p = jnp.transpose(W0.reshape(F, T, C), (1, 0, 2)).reshape(K, C)
    dinv = jnp.asarray(_DINVP).reshape(NB * P2, 2 * C)
    dinv2d = jnp.asarray(_DINV2D)

    out = pl.pallas_call(
        _body,
        grid=(NB,),
        in_specs=[
            pl.BlockSpec(memory_space=pltpu.MemorySpace.HBM),
            pl.BlockSpec(memory_space=pltpu.MemorySpace.HBM),
            pl.BlockSpec((HP, WP), lambda i: (0, 0)),
            pl.BlockSpec((2 * K, 2 * C), lambda i: (0, 0)),
            pl.BlockSpec((1, 2 * C), lambda i: (0, 0)),
            pl.BlockSpec((2 * C, 2 * C), lambda i: (0, 0)),
            pl.BlockSpec((1, 2 * C), lambda i: (0, 0)),
            pl.BlockSpec((2 * C, 2 * C), lambda i: (0, 0)),
            pl.BlockSpec((1, 2 * C), lambda i: (0, 0)),
            pl.BlockSpec((2 * C, 2), lambda i: (0, 0)),
            pl.BlockSpec((1, 1), lambda i: (0, 0)),
        ],
        out_specs=pl.BlockSpec((R, WP), lambda i: (i, 0)),
        out_shape=jax.ShapeDtypeStruct((H, WP), jnp.float32),
        scratch_shapes=[
            pltpu.VMEM((2, NR2, WP, 2 * K), jnp.bfloat16),
            pltpu.VMEM((2, P2, 2 * C), jnp.float32),
            pltpu.SemaphoreType.DMA((2,)),
            pltpu.SemaphoreType.DMA((2,)),
        ],
    )(x3, dinv, dinv2d, _bd(W0p).astype(jnp.bfloat16),
      jnp.tile(b0.reshape(1, C), (1, 2)),
      _bd(Wr0), jnp.tile(br0.reshape(1, C), (1, 2)),
      _bd(Wr1), jnp.tile(br1.reshape(1, C), (1, 2)),
      _bd(W4), b4.reshape(1, 1))

    out = out[:, CPAD:CPAD + W]
    return out.reshape(1, 1, H, W, 1)
